# Initial kernel scaffold; baseline (speedup 1.0000x reference)
import jax
import jax.numpy as jnp
from jax.experimental import pallas as pl


def kernel(x, edge_index, W1, b1, W2, b2, W3, b3):
    raise NotImplementedError("write your pallas kernel here")


# speedup vs baseline: 6.6171x; 6.6171x over previous
import functools

import jax
import jax.numpy as jnp
from jax import lax
from jax.experimental import pallas as pl
from jax.experimental.pallas import tpu as pltpu
from jax.experimental.pallas import tpu_sc as plsc

N = 10000
NP = 10240
E = 160000
D_IN = 256
D_HID = 512
D_OUT = 64

NS = 16
NC = 2
EB = 128
DC = 128
STEPS_A = -(-E // (NS * EB))
STEPS_B = -(-E // (NC * NS * EB))
RPT = NP // NS

F32 = jnp.float32


def _make_agg_chunked(n_chunks: int):
    ch_per_core = n_chunks // NC
    mesh = plsc.VectorSubcoreMesh(core_axis_name="c", subcore_axis_name="s")

    @functools.partial(
        pl.kernel,
        out_type=jax.ShapeDtypeStruct((n_chunks, NP, DC), F32),
        mesh=mesh,
        scratch_types=[
            pltpu.VMEM((STEPS_A, EB), jnp.int32),
            pltpu.VMEM((STEPS_A, EB), jnp.int32),
            pltpu.VMEM((EB, DC), F32),
            pltpu.VMEM_SHARED((NP, DC), F32),
            pltpu.SemaphoreType.DMA,
            pltpu.SemaphoreType.DMA,
        ],
    )
    def agg(g_hbm, src_hbm, dst_hbm, out_hbm, idx_s, idx_d, rows, acc, gsem, ssem):
        cid = lax.axis_index("c")
        sid = lax.axis_index("s")
        pltpu.sync_copy(src_hbm.at[sid], idx_s)
        pltpu.sync_copy(dst_hbm.at[sid], idx_d)
        row0 = sid * RPT
        for q in range(ch_per_core):
            chunk = cid * ch_per_core + q
            gch = g_hbm.at[chunk]
            pltpu.sync_copy(gch.at[pl.ds(row0, RPT)], acc.at[pl.ds(row0, RPT)])
            plsc.subcore_barrier()

            def step(j, carry):
                pltpu.async_copy(gch.at[idx_s.at[j]], rows, gsem).wait()
                pltpu.async_copy(rows, acc.at[idx_d.at[j]], ssem, add=True).wait()
                return carry

            lax.fori_loop(0, STEPS_A, step, 0)
            plsc.subcore_barrier()
            pltpu.sync_copy(acc.at[pl.ds(row0, RPT)],
                            out_hbm.at[chunk].at[pl.ds(row0, RPT)])
            plsc.subcore_barrier()

    return agg


def _make_agg_split():
    mesh = plsc.VectorSubcoreMesh(core_axis_name="c", subcore_axis_name="s")

    @functools.partial(
        pl.kernel,
        out_type=jax.ShapeDtypeStruct((NC, NP, DC), F32),
        mesh=mesh,
        scratch_types=[
            pltpu.VMEM((STEPS_B, EB), jnp.int32),
            pltpu.VMEM((STEPS_B, EB), jnp.int32),
            pltpu.VMEM((EB, DC), F32),
            pltpu.VMEM_SHARED((NP, DC), F32),
            pltpu.SemaphoreType.DMA,
            pltpu.SemaphoreType.DMA,
        ],
    )
    def agg(g_hbm, src_hbm, dst_hbm, out_hbm, idx_s, idx_d, rows, acc, gsem, ssem):
        cid = lax.axis_index("c")
        sid = lax.axis_index("s")
        tid = cid * NS + sid
        pltpu.sync_copy(src_hbm.at[tid], idx_s)
        pltpu.sync_copy(dst_hbm.at[tid], idx_d)
        row0 = sid * RPT
        gch = g_hbm.at[0]
        pltpu.sync_copy(gch.at[pl.ds(row0, RPT)], acc.at[pl.ds(row0, RPT)])
        plsc.subcore_barrier()

        def step(j, carry):
            pltpu.async_copy(gch.at[idx_s.at[j]], rows, gsem).wait()
            pltpu.async_copy(rows, acc.at[idx_d.at[j]], ssem, add=True).wait()
            return carry

        lax.fori_loop(0, STEPS_B, step, 0)
        plsc.subcore_barrier()
        pltpu.sync_copy(acc.at[pl.ds(row0, RPT)],
                        out_hbm.at[cid].at[pl.ds(row0, RPT)])

    return agg


_agg_256 = _make_agg_chunked(2)
_agg_512 = _make_agg_chunked(4)
_agg_split = _make_agg_split()

BM = 1024
_PREC = lax.Precision.HIGHEST


def _tc1_body(degp_ref, x_ref, g0_ref, inv_ref):
    i = pl.program_id(0)
    row = lax.broadcasted_iota(jnp.int32, (BM, 1), 0) + i * BM
    deg = degp_ref[0, :, 0:1] + degp_ref[1, :, 0:1] - 1.0
    inv = jnp.where(row < N, lax.rsqrt(deg), 0.0)
    g0 = inv * x_ref[...]
    g0_ref[0] = g0[:, :128]
    g0_ref[1] = g0[:, 128:]
    inv_ref[...] = jnp.broadcast_to(inv, (BM, 8))


def _tc2_body(a_ref, inv_ref, w_ref, b_ref, out_ref):
    inv = inv_ref[:, 0:1]
    acc = jnp.dot(a_ref[0], w_ref[pl.ds(0, 128), :], precision=_PREC)
    acc = acc + jnp.dot(a_ref[1], w_ref[pl.ds(128, 128), :], precision=_PREC)
    g = inv * jnp.maximum(inv * acc + b_ref[...], 0.0)
    for c in range(4):
        out_ref[c] = g[:, c * 128:(c + 1) * 128]


def _tc3_body(a_ref, inv_ref, w2_ref, b2_ref, w3_ref, out_ref):
    inv = inv_ref[:, 0:1]
    acc = jnp.dot(a_ref[0], w2_ref[pl.ds(0, 128), :], precision=_PREC)
    for c in range(1, 4):
        acc = acc + jnp.dot(a_ref[c], w2_ref[pl.ds(c * 128, 128), :],
                            precision=_PREC)
    h = jnp.maximum(inv * acc + b2_ref[...], 0.0)
    g = inv * jnp.dot(h, w3_ref[...], precision=_PREC)
    out_ref[0] = jnp.concatenate([g, jnp.zeros((BM, DC - D_OUT), F32)], axis=1)


def _tc4_body(a_ref, g2_ref, inv_ref, b_ref, out_ref):
    inv = inv_ref[:, 0:1]
    agg = a_ref[0, :, :D_OUT] + a_ref[1, :, :D_OUT] - g2_ref[0, :, :D_OUT]
    out_ref[...] = inv * agg + b_ref[...]


def kernel(x, edge_index, W1, b1, W2, b2, W3, b3):
    src = edge_index[0].astype(jnp.int32)
    dst = edge_index[1].astype(jnp.int32)
    padA = jnp.full((NS * STEPS_A * EB - E,), N, jnp.int32)
    srcA = jnp.concatenate([src, padA]).reshape(NS, STEPS_A, EB)
    dstA = jnp.concatenate([dst, padA]).reshape(NS, STEPS_A, EB)
    padB = jnp.full((NC * NS * STEPS_B * EB - E,), N, jnp.int32)
    srcB = jnp.concatenate([src, padB]).reshape(NC * NS, STEPS_B, EB)
    dstB = jnp.concatenate([dst, padB]).reshape(NC * NS, STEPS_B, EB)
    x_pad = jnp.pad(x, ((0, NP - N), (0, 0)))
    row_real = (jnp.arange(NP, dtype=jnp.int32) < N).astype(F32)
    ones_c = jnp.broadcast_to(row_real[None, :, None], (1, NP, DC))
    b1r = b1.reshape(1, D_HID)
    b2r = b2.reshape(1, D_HID)
    b3r = b3.reshape(1, D_OUT)

    deg_p = _agg_split(ones_c, srcB, dstB)

    g0c, inv8 = pl.pallas_call(
        _tc1_body,
        grid=(NP // BM,),
        in_specs=[
            pl.BlockSpec((2, BM, DC), lambda i: (0, i, 0)),
            pl.BlockSpec((BM, D_IN), lambda i: (i, 0)),
        ],
        out_specs=[
            pl.BlockSpec((2, BM, 128), lambda i: (0, i, 0)),
            pl.BlockSpec((BM, 8), lambda i: (i, 0)),
        ],
        out_shape=[
            jax.ShapeDtypeStruct((2, NP, 128), F32),
            jax.ShapeDtypeStruct((NP, 8), F32),
        ],
    )(deg_p, x_pad)

    a1c = _agg_256(g0c, srcA, dstA)
    g1c = pl.pallas_call(
        _tc2_body,
        grid=(NP // BM,),
        in_specs=[
            pl.BlockSpec((2, BM, 128), lambda i: (0, i, 0)),
            pl.BlockSpec((BM, 8), lambda i: (i, 0)),
            pl.BlockSpec((D_IN, D_HID), lambda i: (0, 0)),
            pl.BlockSpec((1, D_HID), lambda i: (0, 0)),
        ],
        out_specs=pl.BlockSpec((4, BM, 128), lambda i: (0, i, 0)),
        out_shape=jax.ShapeDtypeStruct((4, NP, 128), F32),
    )(a1c, inv8, W1, b1r)

    a2c = _agg_512(g1c, srcA, dstA)
    g2c = pl.pallas_call(
        _tc3_body,
        grid=(NP // BM,),
        in_specs=[
            pl.BlockSpec((4, BM, 128), lambda i: (0, i, 0)),
            pl.BlockSpec((BM, 8), lambda i: (i, 0)),
            pl.BlockSpec((D_HID, D_HID), lambda i: (0, 0)),
            pl.BlockSpec((1, D_HID), lambda i: (0, 0)),
            pl.BlockSpec((D_HID, D_OUT), lambda i: (0, 0)),
        ],
        out_specs=pl.BlockSpec((1, BM, DC), lambda i: (0, i, 0)),
        out_shape=jax.ShapeDtypeStruct((1, NP, DC), F32),
    )(a2c, inv8, W2, b2r, W3)

    a3p = _agg_split(g2c, srcB, dstB)
    BM4 = 2000
    out = pl.pallas_call(
        _tc4_body,
        grid=(N // BM4,),
        in_specs=[
            pl.BlockSpec((2, BM4, DC), lambda i: (0, i, 0)),
            pl.BlockSpec((1, BM4, DC), lambda i: (0, i, 0)),
            pl.BlockSpec((BM4, 8), lambda i: (i, 0)),
            pl.BlockSpec((1, D_OUT), lambda i: (0, 0)),
        ],
        out_specs=pl.BlockSpec((BM4, D_OUT), lambda i: (i, 0)),
        out_shape=jax.ShapeDtypeStruct((N, D_OUT), F32),
    )(a3p, g2c, inv8, b3r)
    return out
